# baseline (speedup 1.0000x reference)
import functools

import jax
import jax.numpy as jnp
from jax import lax
from jax.experimental import pallas as pl
from jax.experimental.pallas import tpu as pltpu
from jax.experimental.pallas import tpu_sc as plsc

N_TOK = 32768
DIM = 256
K = 256
BLK = 2048
NSLICE = 2
TOK_S = N_TOK // NSLICE

NC = 2
NS = 16
NW = NC * NS
CHUNK = 128


def _tc_body(x_ref, w_ref, idx_ref):
    x = x_ref[...]
    w = w_ref[...]
    wt2 = jnp.sum(w * w, axis=0, keepdims=True)
    x2 = jnp.sum(x * x, axis=1, keepdims=True)
    cross = jax.lax.dot_general(
        x, w, (((1,), (0,)), ((), ())),
        preferred_element_type=jnp.float32,
    )
    dist = x2 + wt2 - 2.0 * cross
    m = jnp.min(dist, axis=1, keepdims=True)
    iota = jax.lax.broadcasted_iota(jnp.int32, dist.shape, 1).astype(jnp.float32)
    idx = jnp.min(jnp.where(dist == m, iota, float(K)), axis=1, keepdims=True)
    idx_ref[...] = jnp.reshape(idx.astype(jnp.int32), (BLK // 128, 128))


def _tc_indices(x_slice, W, ntok):
    return pl.pallas_call(
        _tc_body,
        grid=(ntok // BLK,),
        in_specs=[
            pl.BlockSpec((BLK, DIM), lambda i: (i, 0)),
            pl.BlockSpec((DIM, K), lambda i: (0, 0)),
        ],
        out_specs=pl.BlockSpec((BLK // 128, 128), lambda i: (i, 0)),
        out_shape=jax.ShapeDtypeStruct((ntok // 128, 128), jnp.int32),
    )(x_slice, W)


_sc_mesh = plsc.VectorSubcoreMesh(core_axis_name="c", subcore_axis_name="s")


def _make_sc_gather(ntok):
    b_per_w = ntok // NW
    nchunk = b_per_w // CHUNK

    @functools.partial(
        pl.kernel,
        out_type=jax.ShapeDtypeStruct((ntok, DIM), jnp.float32),
        mesh=_sc_mesh,
        scratch_types=[
            pltpu.VMEM((nchunk, CHUNK), jnp.int32),
            pltpu.VMEM((CHUNK, DIM), jnp.float32),
            pltpu.VMEM((CHUNK, DIM), jnp.float32),
            pltpu.SemaphoreType.DMA,
            pltpu.SemaphoreType.DMA,
        ],
        compiler_params=pltpu.CompilerParams(needs_layout_passes=False),
    )
    def sc_gather(w_hbm, idx_hbm, out_hbm, idx_v, buf0, buf1, gsem, ssem):
        wid = lax.axis_index("s") * NC + lax.axis_index("c")
        base = wid * b_per_w
        pltpu.sync_copy(idx_hbm.at[pl.ds(wid * nchunk, nchunk)], idx_v)
        bufs = (buf0, buf1)
        gathers = [None] * nchunk
        stores = [None] * nchunk
        gathers[0] = pltpu.async_copy(w_hbm.at[idx_v.at[0]], bufs[0], gsem)
        for c in range(nchunk):
            gathers[c].wait()
            if c + 1 < nchunk:
                if c >= 1:
                    stores[c - 1].wait()
                gathers[c + 1] = pltpu.async_copy(
                    w_hbm.at[idx_v.at[c + 1]], bufs[(c + 1) % 2], gsem)
            stores[c] = pltpu.async_copy(
                bufs[c % 2], out_hbm.at[pl.ds(base + c * CHUNK, CHUNK)], ssem)
        stores[nchunk - 1].wait()

    return sc_gather


_sc_gather_slice = _make_sc_gather(TOK_S)


@jax.jit
def kernel(x_flat, W):
    outs = []
    for s in range(NSLICE):
        x_s = lax.slice(x_flat, (s * TOK_S, 0), ((s + 1) * TOK_S, DIM))
        idx_s = _tc_indices(x_s, W, TOK_S)
        outs.append(_sc_gather_slice(W, idx_s))
    return jnp.concatenate(outs, axis=0)


# speedup vs baseline: 1.1315x; 1.1315x over previous
import functools

import jax
import jax.numpy as jnp
from jax import lax
from jax.experimental import pallas as pl
from jax.experimental.pallas import tpu as pltpu
from jax.experimental.pallas import tpu_sc as plsc

N_TOK = 32768
DIM = 256
K = 256
BLK = 2048

NC = 2
NS = 16
NW = NC * NS
B_PER_W = N_TOK // NW
CHUNK = 128
NCHUNK = B_PER_W // CHUNK
CH2 = 64
NCH2 = B_PER_W // CH2


def _tc_body(x_ref, w_ref, idx_ref):
    x = x_ref[...]
    w = w_ref[...]
    wt2 = jnp.sum(w * w, axis=0, keepdims=True)
    x2 = jnp.sum(x * x, axis=1, keepdims=True)
    cross = jax.lax.dot_general(
        x, w, (((1,), (0,)), ((), ())),
        preferred_element_type=jnp.float32,
    )
    dist = x2 + wt2 - 2.0 * cross
    m = jnp.min(dist, axis=1, keepdims=True)
    iota = jax.lax.broadcasted_iota(jnp.int32, dist.shape, 1).astype(jnp.float32)
    idx = jnp.min(jnp.where(dist == m, iota, float(K)), axis=1, keepdims=True)
    idx_ref[...] = jnp.reshape(idx.astype(jnp.int32), (BLK // 128, 128))


def _tc_indices(x_flat, W):
    grid = (N_TOK // BLK,)
    return pl.pallas_call(
        _tc_body,
        grid=grid,
        in_specs=[
            pl.BlockSpec((BLK, DIM), lambda i: (i, 0)),
            pl.BlockSpec((DIM, K), lambda i: (0, 0)),
        ],
        out_specs=pl.BlockSpec((BLK // 128, 128), lambda i: (i, 0)),
        out_shape=jax.ShapeDtypeStruct((N_TOK // 128, 128), jnp.int32),
    )(x_flat, W)


_sc_mesh = plsc.VectorSubcoreMesh(core_axis_name="c", subcore_axis_name="s")


WPAD = DIM + 8


@functools.partial(
    pl.kernel,
    out_type=jax.ShapeDtypeStruct((N_TOK * DIM,), jnp.float32),
    mesh=_sc_mesh,
    scratch_types=[
        pltpu.VMEM((B_PER_W,), jnp.int32),
        pltpu.VMEM((K * WPAD,), jnp.float32),
        pltpu.VMEM((CH2 * DIM,), jnp.float32),
        pltpu.VMEM((CH2 * DIM,), jnp.float32),
        pltpu.SemaphoreType.DMA,
    ],
    compiler_params=pltpu.CompilerParams(needs_layout_passes=False),
)
def _sc_gather(w_hbm, idx_hbm, out_hbm, idx_v, w_v, buf0, buf1, ssem):
    wid = lax.axis_index("s") * NC + lax.axis_index("c")
    base = wid * B_PER_W
    pltpu.sync_copy(w_hbm, w_v)
    pltpu.sync_copy(idx_hbm.at[pl.ds(base, B_PER_W)], idx_v)
    bufs = (buf0, buf1)
    lanes = jax.lax.iota(jnp.int32, 16)
    zeros = lanes * 0
    cols = [(lanes + j * 16) * WPAD for j in range(DIM // 16)]
    stores = [None] * NCH2

    def build(c, buf):
        @plsc.parallel_loop(0, CH2, unroll=4)
        def body(t):
            tsplat = plsc.load_gather(idx_v, [zeros + (c * CH2 + t)])
            for j in range(DIM // 16):
                v = plsc.load_gather(w_v, [tsplat + cols[j]])
                buf[pl.ds(t * DIM + j * 16, 16)] = v

    for c in range(NCH2):
        if c >= 2:
            stores[c - 2].wait()
        build(c, bufs[c % 2])
        stores[c] = pltpu.async_copy(
            bufs[c % 2], out_hbm.at[pl.ds((base + c * CH2) * DIM, CH2 * DIM)],
            ssem)
    stores[NCH2 - 2].wait()
    stores[NCH2 - 1].wait()


@jax.jit
def kernel(x_flat, W):
    idx = _tc_indices(x_flat, W).reshape(-1)
    w_pad = jnp.pad(W.T, ((0, 0), (0, WPAD - DIM))).reshape(-1)
    return _sc_gather(w_pad, idx).reshape(N_TOK, DIM)


# speedup vs baseline: 1.5231x; 1.3461x over previous
import functools

import jax
import jax.numpy as jnp
from jax import lax
from jax.experimental import pallas as pl
from jax.experimental.pallas import tpu as pltpu
from jax.experimental.pallas import tpu_sc as plsc

N_TOK = 32768
DIM = 256
K = 256
BLK = 4096

NC = 2
NS = 16
NW = NC * NS
B_PER_W = N_TOK // NW
CHUNK = 128
NCHUNK = B_PER_W // CHUNK


def _tc_body(x_ref, w_ref, idx_ref):
    x = x_ref[...]
    w = w_ref[...]
    wt2 = jnp.sum(w * w, axis=0, keepdims=True)
    x2 = jnp.sum(x * x, axis=1, keepdims=True)
    cross = jax.lax.dot_general(
        x, w, (((1,), (0,)), ((), ())),
        preferred_element_type=jnp.float32,
    )
    dist = x2 + wt2 - 2.0 * cross
    m = jnp.min(dist, axis=1, keepdims=True)
    iota = jax.lax.broadcasted_iota(jnp.int32, dist.shape, 1).astype(jnp.float32)
    idx = jnp.min(jnp.where(dist == m, iota, float(K)), axis=1, keepdims=True)
    idx_ref[...] = jnp.reshape(idx.astype(jnp.int32), (BLK // 128, 128))


def _tc_indices(x_flat, W):
    grid = (N_TOK // BLK,)
    return pl.pallas_call(
        _tc_body,
        grid=grid,
        in_specs=[
            pl.BlockSpec((BLK, DIM), lambda i: (i, 0)),
            pl.BlockSpec((DIM, K), lambda i: (0, 0)),
        ],
        out_specs=pl.BlockSpec((BLK // 128, 128), lambda i: (i, 0)),
        out_shape=jax.ShapeDtypeStruct((N_TOK // 128, 128), jnp.int32),
    )(x_flat, W)


_sc_mesh = plsc.VectorSubcoreMesh(core_axis_name="c", subcore_axis_name="s")


@functools.partial(
    pl.kernel,
    out_type=jax.ShapeDtypeStruct((N_TOK, DIM), jnp.float32),
    mesh=_sc_mesh,
    scratch_types=[
        pltpu.VMEM((NCHUNK, CHUNK), jnp.int32),
        pltpu.VMEM((CHUNK, DIM), jnp.float32),
        pltpu.VMEM((CHUNK, DIM), jnp.float32),
        pltpu.SemaphoreType.DMA,
        pltpu.SemaphoreType.DMA,
    ],
)
def _sc_gather(w_hbm, idx_hbm, out_hbm, idx_v, buf0, buf1, gsem, ssem):
    wid = lax.axis_index("s") * NC + lax.axis_index("c")
    base = wid * B_PER_W
    pltpu.sync_copy(idx_hbm.at[pl.ds(wid * NCHUNK, NCHUNK)], idx_v)
    bufs = (buf0, buf1)
    gathers = [None] * NCHUNK
    stores = [None] * NCHUNK
    gathers[0] = pltpu.async_copy(w_hbm.at[idx_v.at[0]], bufs[0], gsem)
    for c in range(NCHUNK):
        gathers[c].wait()
        if c + 1 < NCHUNK:
            if c >= 1:
                stores[c - 1].wait()
            gathers[c + 1] = pltpu.async_copy(
                w_hbm.at[idx_v.at[c + 1]], bufs[(c + 1) % 2], gsem)
        stores[c] = pltpu.async_copy(
            bufs[c % 2], out_hbm.at[pl.ds(base + c * CHUNK, CHUNK)], ssem)
    stores[NCHUNK - 1].wait()


@jax.jit
def kernel(x_flat, W):
    idx = _tc_indices(x_flat, W)
    return _sc_gather(W, idx)
